# Initial kernel scaffold; baseline (speedup 1.0000x reference)
import functools

import jax
import jax.numpy as jnp
from jax import lax
from jax.experimental import pallas as pl
from jax.experimental.pallas import tpu as pltpu
from jax.experimental.pallas import tpu_sc as plsc

_EPS = 1e-5
_K1 = 16
_K2 = 12
_BQ = 256
_BLK = 256


def _stage1_body(x_ref, w_ref, g_ref, b_ref, y_ref, ytp_ref, sq_ref):
    x = x_ref[0]
    w = w_ref[...]
    y = lax.dot_general(w, x, (((1,), (0,)), ((), ())),
                        preferred_element_type=jnp.float32)
    scale = g_ref[...] / jnp.sqrt(1.0 + _EPS)
    y = y * scale + b_ref[...]
    y = jnp.maximum(y, 0.0)
    y_ref[0] = y
    sq_ref[0] = jnp.sum(y * y, axis=0, keepdims=True)
    yt = y.T
    ytp_ref[0] = jnp.concatenate([yt, jnp.zeros_like(yt)], axis=1)


def _knn_body(ytp_ref, y_ref, sq_ref, idx_ref):
    n = y_ref.shape[2]
    b = pl.program_id(0)
    yq = ytp_ref[0][:, :8]
    yk = y_ref[0]
    sqk = sq_ref[0]
    d = 2.0 * lax.dot_general(yq, yk, (((1,), (0,)), ((), ())),
                              preferred_element_type=jnp.float32)
    d = d - sqk
    iota = lax.broadcasted_iota(jnp.int32, d.shape, 1)
    cols = []
    for _ in range(_K1):
        m = jnp.max(d, axis=1, keepdims=True)
        cand = jnp.where(d == m, iota, n)
        aj = jnp.min(cand, axis=1, keepdims=True)
        cols.append(aj)
        d = jnp.where(iota == aj, -jnp.inf, d)
    idx = jnp.concatenate(cols, axis=1)
    idx_ref[0] = idx + b * n


def _branch(gb, y_i, blk, w_at, w_dt, b1, w2t, b2, wct, wpt, bp, kmax):
    n1 = jnp.dot(gb, w_at, preferred_element_type=jnp.float32)
    q1 = jnp.dot(y_i, w_dt, preferred_element_type=jnp.float32) + b1
    h1 = jnp.maximum(n1.reshape(blk, _K1, 64) + q1[:, None, :], 0.0)
    h1 = h1.reshape(blk * _K1, 64)
    h2 = jnp.maximum(jnp.dot(h1, w2t, preferred_element_type=jnp.float32) + b2,
                     0.0)
    s = jax.nn.sigmoid(jnp.dot(h2, wct, preferred_element_type=jnp.float32))
    ps = (h2 * s).reshape(blk, _K1, 128)
    if kmax < _K1:
        slot = lax.broadcasted_iota(jnp.int32, ps.shape, 1)
        ps = jnp.where(slot < kmax, ps, 0.0)
    pooled = jnp.sum(ps, axis=1)
    return jnp.maximum(jnp.dot(pooled, wpt,
                               preferred_element_type=jnp.float32) + bp, 0.0)


def _fuse_body(g_ref, ytp_ref,
               w1at_ref, w1dt_ref, b1_ref, w2t_ref, b2_ref, wc1_ref,
               wp1_ref, bp1_ref,
               w3at_ref, w3dt_ref, b3_ref, w4t_ref, b4_ref, wc2_ref,
               wp2_ref, bp2_ref,
               wft_ref, bf_ref, out_ref):
    gb = g_ref[0][:, :8]
    y_i = ytp_ref[0][:, :8]
    x1 = _branch(gb, y_i, _BLK, w1at_ref[...], w1dt_ref[...], b1_ref[...],
                 w2t_ref[...], b2_ref[...], wc1_ref[...], wp1_ref[...],
                 bp1_ref[...], _K1)
    x2 = _branch(gb, y_i, _BLK, w3at_ref[...], w3dt_ref[...], b3_ref[...],
                 w4t_ref[...], b4_ref[...], wc2_ref[...], wp2_ref[...],
                 bp2_ref[...], _K2)
    xcat = jnp.concatenate([x1, x2], axis=1)
    out = jnp.maximum(jnp.dot(xcat, wft_ref[...],
                              preferred_element_type=jnp.float32)
                      + bf_ref[...], 0.0)
    out_ref[0] = out.T


def _sc_gather(table, idx):
    m = idx.shape[0]
    nw = 32
    bpw = m // nw
    mesh = plsc.VectorSubcoreMesh(core_axis_name="c", subcore_axis_name="s")

    @functools.partial(
        pl.kernel, mesh=mesh,
        out_type=jax.ShapeDtypeStruct((m, 16), jnp.float32),
        scratch_types=[
            pltpu.VMEM((bpw,), jnp.int32),
            pltpu.VMEM((bpw, 16), jnp.float32),
            pltpu.SemaphoreType.DMA,
        ],
    )
    def gather_k(table_hbm, idx_hbm, out_hbm, idx_v, rows_v, sem):
        wid = lax.axis_index("s") * 2 + lax.axis_index("c")
        base = wid * bpw
        pltpu.sync_copy(idx_hbm.at[pl.ds(base, bpw)], idx_v)
        pltpu.async_copy(table_hbm.at[idx_v], rows_v, sem).wait()
        pltpu.sync_copy(rows_v, out_hbm.at[pl.ds(base, bpw)])

    return gather_k(table, idx)


def _fold(w, g, b):
    s = g / jnp.sqrt(1.0 + _EPS)
    return (w * s[:, None]).T, b[None, :]


def kernel(x, params):
    p = params
    bsz, _, n = x.shape

    g1 = p['liner_g'].reshape(8, 1)
    b1 = p['liner_b'].reshape(8, 1)
    y, ytp, sq = pl.pallas_call(
        _stage1_body,
        grid=(bsz,),
        in_specs=[
            pl.BlockSpec((1, 3, n), lambda b: (b, 0, 0)),
            pl.BlockSpec((8, 3), lambda b: (0, 0)),
            pl.BlockSpec((8, 1), lambda b: (0, 0)),
            pl.BlockSpec((8, 1), lambda b: (0, 0)),
        ],
        out_specs=[
            pl.BlockSpec((1, 8, n), lambda b: (b, 0, 0)),
            pl.BlockSpec((1, n, 16), lambda b: (b, 0, 0)),
            pl.BlockSpec((1, 1, n), lambda b: (b, 0, 0)),
        ],
        out_shape=[
            jax.ShapeDtypeStruct((bsz, 8, n), jnp.float32),
            jax.ShapeDtypeStruct((bsz, n, 16), jnp.float32),
            jax.ShapeDtypeStruct((bsz, 1, n), jnp.float32),
        ],
    )(x, p['liner_W'], g1, b1)

    idxg = pl.pallas_call(
        _knn_body,
        grid=(bsz, n // _BQ),
        in_specs=[
            pl.BlockSpec((1, _BQ, 16), lambda b, q: (b, q, 0)),
            pl.BlockSpec((1, 8, n), lambda b, q: (b, 0, 0)),
            pl.BlockSpec((1, 1, n), lambda b, q: (b, 0, 0)),
        ],
        out_specs=pl.BlockSpec((1, _BQ, _K1), lambda b, q: (b, q, 0)),
        out_shape=jax.ShapeDtypeStruct((bsz, n, _K1), jnp.int32),
    )(ytp, y, sq)

    table = ytp.reshape(bsz * n, 16)
    gathered = _sc_gather(table, idxg.reshape(bsz * n * _K1))
    gr = gathered.reshape(bsz, n * _K1, 16)

    w1t, bb1 = _fold(p['conv1_W'], p['bn1_g'], p['bn1_b'])
    w2t, bb2 = _fold(p['conv2_W'], p['bn2_g'], p['bn2_b'])
    w3t, bb3 = _fold(p['conv3_W'], p['bn3_g'], p['bn3_b'])
    w4t, bb4 = _fold(p['conv4_W'], p['bn4_g'], p['bn4_b'])
    wp1, bp1 = _fold(p['att1_W2'], p['att1_g'], p['att1_b'])
    wp2, bp2 = _fold(p['att2_W2'], p['att2_g'], p['att2_b'])
    wft, bf = _fold(p['conv_W'], p['bn_g'], p['bn_b'])
    wc1 = p['att1_Wc'].T
    wc2 = p['att2_Wc'].T
    w1at, w1dt = w1t[:8], w1t[8:] - w1t[:8]
    w3at, w3dt = w3t[:8], w3t[8:] - w3t[:8]

    def wspec(shape):
        return pl.BlockSpec(shape, lambda b, q: tuple(0 for _ in shape))

    out = pl.pallas_call(
        _fuse_body,
        grid=(bsz, n // _BLK),
        in_specs=[
            pl.BlockSpec((1, _BLK * _K1, 16), lambda b, q: (b, q, 0)),
            pl.BlockSpec((1, _BLK, 16), lambda b, q: (b, q, 0)),
            wspec((8, 64)), wspec((8, 64)), wspec((1, 64)),
            wspec((64, 128)), wspec((1, 128)), wspec((128, 128)),
            wspec((128, 256)), wspec((1, 256)),
            wspec((8, 64)), wspec((8, 64)), wspec((1, 64)),
            wspec((64, 128)), wspec((1, 128)), wspec((128, 128)),
            wspec((128, 256)), wspec((1, 256)),
            wspec((512, 512)), wspec((1, 512)),
        ],
        out_specs=pl.BlockSpec((1, 512, _BLK), lambda b, q: (b, 0, q)),
        out_shape=jax.ShapeDtypeStruct((bsz, 512, n), jnp.float32),
    )(gr, ytp,
      w1at, w1dt, bb1, w2t, bb2, wc1, wp1, bp1,
      w3at, w3dt, bb3, w4t, bb4, wc2, wp2, bp2,
      wft, bf)
    return out


# speedup vs baseline: 13.9774x; 13.9774x over previous
import functools

import jax
import jax.numpy as jnp
from jax import lax
from jax.experimental import pallas as pl
from jax.experimental.pallas import tpu as pltpu
from jax.experimental.pallas import tpu_sc as plsc

_EPS = 1e-5
_K1 = 16
_K2 = 12
_BQ = 256
_BLK = 256


def _stage1_body(x_ref, w_ref, g_ref, b_ref,
                 w1at_ref, w1dt_ref, bb1_ref, w3at_ref, w3dt_ref, bb3_ref,
                 y_ref, ytp_ref, sq_ref, ptab_ref, qtab_ref):
    x = x_ref[0]
    w = w_ref[...]
    y = lax.dot_general(w, x, (((1,), (0,)), ((), ())),
                        preferred_element_type=jnp.float32)
    scale = g_ref[...] / jnp.sqrt(1.0 + _EPS)
    y = y * scale + b_ref[...]
    y = jnp.maximum(y, 0.0)
    y_ref[0] = y
    sq_ref[0] = jnp.sum(y * y, axis=0, keepdims=True)
    yt = y.T
    ytp_ref[0] = jnp.concatenate([yt, jnp.zeros_like(yt)], axis=1)
    p1 = jnp.dot(yt, w1at_ref[...], preferred_element_type=jnp.float32)
    p3 = jnp.dot(yt, w3at_ref[...], preferred_element_type=jnp.float32)
    ptab_ref[0] = jnp.concatenate([p1, p3], axis=1)
    q1 = jnp.dot(yt, w1dt_ref[...], preferred_element_type=jnp.float32)
    q3 = jnp.dot(yt, w3dt_ref[...], preferred_element_type=jnp.float32)
    qtab_ref[0] = jnp.concatenate([q1 + bb1_ref[...], q3 + bb3_ref[...]],
                                  axis=1)


def _knn_body(ytp_ref, y_ref, sq_ref, idx_ref):
    n = y_ref.shape[2]
    b = pl.program_id(0)
    yq = ytp_ref[0][:, :8]
    yk = y_ref[0]
    sqk = sq_ref[0]
    d = 2.0 * lax.dot_general(yq, yk, (((1,), (0,)), ((), ())),
                              preferred_element_type=jnp.float32)
    d = d - sqk
    iota = lax.broadcasted_iota(jnp.int32, d.shape, 1)
    cols = []
    for _ in range(_K1):
        m = jnp.max(d, axis=1, keepdims=True)
        cand = jnp.where(d == m, iota, n)
        aj = jnp.min(cand, axis=1, keepdims=True)
        cols.append(aj)
        d = jnp.where(iota == aj, -jnp.inf, d)
    idx = jnp.concatenate(cols, axis=1)
    idx_ref[0] = idx + b * n


def _branch(n1, q1, blk, w2t, b2, wct, wpt, bp, kmax):
    h1 = jnp.maximum(n1.reshape(blk, _K1, 64) + q1[:, None, :], 0.0)
    h1 = h1.reshape(blk * _K1, 64)
    h2 = jnp.maximum(jnp.dot(h1, w2t, preferred_element_type=jnp.float32) + b2,
                     0.0)
    s = jax.nn.sigmoid(jnp.dot(h2, wct, preferred_element_type=jnp.float32))
    ps = (h2 * s).reshape(blk, _K1, 128)
    if kmax < _K1:
        slot = lax.broadcasted_iota(jnp.int32, ps.shape, 1)
        ps = jnp.where(slot < kmax, ps, 0.0)
    pooled = jnp.sum(ps, axis=1)
    return jnp.maximum(jnp.dot(pooled, wpt,
                               preferred_element_type=jnp.float32) + bp, 0.0)


def _fuse_body(g_ref, qtab_ref,
               w2t_ref, b2_ref, wc1_ref, wp1_ref, bp1_ref,
               w4t_ref, b4_ref, wc2_ref, wp2_ref, bp2_ref,
               wft_ref, bf_ref, out_ref):
    g = g_ref[0]
    q = qtab_ref[0]
    x1 = _branch(g[:, :64], q[:, :64], _BLK, w2t_ref[...], b2_ref[...],
                 wc1_ref[...], wp1_ref[...], bp1_ref[...], _K1)
    x2 = _branch(g[:, 64:], q[:, 64:], _BLK, w4t_ref[...], b4_ref[...],
                 wc2_ref[...], wp2_ref[...], bp2_ref[...], _K2)
    xcat = jnp.concatenate([x1, x2], axis=1)
    out = jnp.maximum(jnp.dot(xcat, wft_ref[...],
                              preferred_element_type=jnp.float32)
                      + bf_ref[...], 0.0)
    out_ref[0] = out.T


def _sc_gather(table, idx):
    m = idx.shape[0]
    nw = 32
    bpw = m // nw
    ch = 256
    nch = bpw // ch
    mesh = plsc.VectorSubcoreMesh(core_axis_name="c", subcore_axis_name="s")

    @functools.partial(
        pl.kernel, mesh=mesh,
        out_type=jax.ShapeDtypeStruct((m, 128), jnp.float32),
        scratch_types=[
            pltpu.VMEM((bpw,), jnp.int32),
            pltpu.VMEM((ch, 128), jnp.float32),
            pltpu.VMEM((ch, 128), jnp.float32),
            pltpu.SemaphoreType.DMA,
            pltpu.SemaphoreType.DMA,
        ],
    )
    def gather_k(table_hbm, idx_hbm, out_hbm, idx_v, rows_a, rows_b, sem_a,
                 sem_b):
        wid = lax.axis_index("s") * 2 + lax.axis_index("c")
        base = wid * bpw
        pltpu.sync_copy(idx_hbm.at[pl.ds(base, bpw)], idx_v)
        bufs = (rows_a, rows_b)
        sems = (sem_a, sem_b)
        cps = [None, None]
        for c in range(nch):
            p = c % 2
            cps[p] = pltpu.async_copy(
                table_hbm.at[idx_v.at[pl.ds(c * ch, ch)]], bufs[p], sems[p])
            if c > 0:
                q = 1 - p
                cps[q].wait()
                pltpu.sync_copy(bufs[q],
                                out_hbm.at[pl.ds(base + (c - 1) * ch, ch)])
        p = (nch - 1) % 2
        cps[p].wait()
        pltpu.sync_copy(bufs[p], out_hbm.at[pl.ds(base + (nch - 1) * ch, ch)])

    return gather_k(table, idx)


def _fold(w, g, b):
    s = g / jnp.sqrt(1.0 + _EPS)
    return (w * s[:, None]).T, b[None, :]


def kernel(x, params):
    p = params
    bsz, _, n = x.shape

    w1t, bb1 = _fold(p['conv1_W'], p['bn1_g'], p['bn1_b'])
    w2t, bb2 = _fold(p['conv2_W'], p['bn2_g'], p['bn2_b'])
    w3t, bb3 = _fold(p['conv3_W'], p['bn3_g'], p['bn3_b'])
    w4t, bb4 = _fold(p['conv4_W'], p['bn4_g'], p['bn4_b'])
    wp1, bp1 = _fold(p['att1_W2'], p['att1_g'], p['att1_b'])
    wp2, bp2 = _fold(p['att2_W2'], p['att2_g'], p['att2_b'])
    wft, bf = _fold(p['conv_W'], p['bn_g'], p['bn_b'])
    wc1 = p['att1_Wc'].T
    wc2 = p['att2_Wc'].T
    w1at, w1dt = w1t[:8], w1t[8:] - w1t[:8]
    w3at, w3dt = w3t[:8], w3t[8:] - w3t[:8]

    g1 = p['liner_g'].reshape(8, 1)
    b1 = p['liner_b'].reshape(8, 1)
    y, ytp, sq, ptab, qtab = pl.pallas_call(
        _stage1_body,
        grid=(bsz,),
        in_specs=[
            pl.BlockSpec((1, 3, n), lambda b: (b, 0, 0)),
            pl.BlockSpec((8, 3), lambda b: (0, 0)),
            pl.BlockSpec((8, 1), lambda b: (0, 0)),
            pl.BlockSpec((8, 1), lambda b: (0, 0)),
            pl.BlockSpec((8, 64), lambda b: (0, 0)),
            pl.BlockSpec((8, 64), lambda b: (0, 0)),
            pl.BlockSpec((1, 64), lambda b: (0, 0)),
            pl.BlockSpec((8, 64), lambda b: (0, 0)),
            pl.BlockSpec((8, 64), lambda b: (0, 0)),
            pl.BlockSpec((1, 64), lambda b: (0, 0)),
        ],
        out_specs=[
            pl.BlockSpec((1, 8, n), lambda b: (b, 0, 0)),
            pl.BlockSpec((1, n, 16), lambda b: (b, 0, 0)),
            pl.BlockSpec((1, 1, n), lambda b: (b, 0, 0)),
            pl.BlockSpec((1, n, 128), lambda b: (b, 0, 0)),
            pl.BlockSpec((1, n, 128), lambda b: (b, 0, 0)),
        ],
        out_shape=[
            jax.ShapeDtypeStruct((bsz, 8, n), jnp.float32),
            jax.ShapeDtypeStruct((bsz, n, 16), jnp.float32),
            jax.ShapeDtypeStruct((bsz, 1, n), jnp.float32),
            jax.ShapeDtypeStruct((bsz, n, 128), jnp.float32),
            jax.ShapeDtypeStruct((bsz, n, 128), jnp.float32),
        ],
    )(x, p['liner_W'], g1, b1, w1at, w1dt, bb1, w3at, w3dt, bb3)

    idxg = pl.pallas_call(
        _knn_body,
        grid=(bsz, n // _BQ),
        in_specs=[
            pl.BlockSpec((1, _BQ, 16), lambda b, q: (b, q, 0)),
            pl.BlockSpec((1, 8, n), lambda b, q: (b, 0, 0)),
            pl.BlockSpec((1, 1, n), lambda b, q: (b, 0, 0)),
        ],
        out_specs=pl.BlockSpec((1, _BQ, _K1), lambda b, q: (b, q, 0)),
        out_shape=jax.ShapeDtypeStruct((bsz, n, _K1), jnp.int32),
    )(ytp, y, sq)

    gathered = _sc_gather(ptab.reshape(bsz * n, 128),
                          idxg.reshape(bsz * n * _K1))
    gr = gathered.reshape(bsz, n * _K1, 128)

    def wspec(shape):
        return pl.BlockSpec(shape, lambda b, q: tuple(0 for _ in shape))

    out = pl.pallas_call(
        _fuse_body,
        grid=(bsz, n // _BLK),
        in_specs=[
            pl.BlockSpec((1, _BLK * _K1, 128), lambda b, q: (b, q, 0)),
            pl.BlockSpec((1, _BLK, 128), lambda b, q: (b, q, 0)),
            wspec((64, 128)), wspec((1, 128)), wspec((128, 128)),
            wspec((128, 256)), wspec((1, 256)),
            wspec((64, 128)), wspec((1, 128)), wspec((128, 128)),
            wspec((128, 256)), wspec((1, 256)),
            wspec((512, 512)), wspec((1, 512)),
        ],
        out_specs=pl.BlockSpec((1, 512, _BLK), lambda b, q: (b, 0, q)),
        out_shape=jax.ShapeDtypeStruct((bsz, 512, n), jnp.float32),
    )(gr, qtab,
      w2t, bb2, wc1, wp1, bp1,
      w4t, bb4, wc2, wp2, bp2,
      wft, bf)
    return out


# speedup vs baseline: 16.8222x; 1.2035x over previous
import functools

import jax
import jax.numpy as jnp
from jax import lax
from jax.experimental import pallas as pl
from jax.experimental.pallas import tpu as pltpu
from jax.experimental.pallas import tpu_sc as plsc

_EPS = 1e-5
_K1 = 16
_K2 = 12
_BQ = 256
_BLK = 256


def _dgt(a, b):
    return lax.dot_general(a, b, (((1,), (1,)), ((), ())),
                           preferred_element_type=jnp.float32)


def _stage1_body(x_ref, w_ref, g_ref, b_ref,
                 w1a_ref, w1d_ref, s1_ref, bb1_ref,
                 w3a_ref, w3d_ref, s3_ref, bb3_ref,
                 y_ref, ytp_ref, sq_ref, ptab_ref, qtab_ref):
    x = x_ref[0]
    w = w_ref[...]
    y = lax.dot_general(w, x, (((1,), (0,)), ((), ())),
                        preferred_element_type=jnp.float32)
    scale = g_ref[...] / jnp.sqrt(1.0 + _EPS)
    y = y * scale + b_ref[...]
    y = jnp.maximum(y, 0.0)
    y_ref[0] = y
    sq_ref[0] = jnp.sum(y * y, axis=0, keepdims=True)
    yt = y.T
    ytp_ref[0] = jnp.concatenate([yt, jnp.zeros_like(yt)], axis=1)
    s1 = s1_ref[...]
    s3 = s3_ref[...]
    p1 = _dgt(yt, w1a_ref[...]) * s1
    p3 = _dgt(yt, w3a_ref[...]) * s3
    ptab_ref[0] = jnp.concatenate([p1, p3], axis=1)
    q1 = _dgt(yt, w1d_ref[...]) * s1 + bb1_ref[...]
    q3 = _dgt(yt, w3d_ref[...]) * s3 + bb3_ref[...]
    qtab_ref[0] = jnp.concatenate([q1, q3], axis=1)


def _knn_body(base, ytp_ref, y_ref, sq_ref, idx_ref):
    n = y_ref.shape[2]
    yq = ytp_ref[0][:, :8]
    yk = y_ref[0]
    sqk = sq_ref[0]
    d = 2.0 * lax.dot_general(yq, yk, (((1,), (0,)), ((), ())),
                              preferred_element_type=jnp.float32)
    d = d - sqk
    niota = (float(n)
             - lax.broadcasted_iota(jnp.int32, d.shape, 1).astype(jnp.float32))
    cols = []
    for _ in range(_K1):
        m = jnp.max(d, axis=1, keepdims=True)
        c = jnp.where(d == m, niota, 0.0)
        mx = jnp.max(c, axis=1, keepdims=True)
        cols.append(mx)
        d = jnp.where(niota == mx, -jnp.inf, d)
    idxf = float(n) - jnp.concatenate(cols, axis=1)
    idx_ref[...] = idxf.astype(jnp.int32) + base


def _branch(n1, q1, blk, w2, s2, b2, wc, wp, sp, bp, kmax):
    h1 = jnp.maximum(n1.reshape(blk, _K1, 64) + q1[:, None, :], 0.0)
    h1 = h1.reshape(blk * _K1, 64)
    h2 = jnp.maximum(_dgt(h1, w2) * s2 + b2, 0.0)
    s = jax.nn.sigmoid(_dgt(h2, wc))
    ps = (h2 * s).reshape(blk, _K1, 128)
    if kmax < _K1:
        slot = lax.broadcasted_iota(jnp.int32, ps.shape, 1)
        ps = jnp.where(slot < kmax, ps, 0.0)
    pooled = jnp.sum(ps, axis=1)
    return jnp.maximum(_dgt(pooled, wp) * sp + bp, 0.0)


def _fuse_body(g_ref, qtab_ref,
               w2_ref, s2_ref, b2_ref, wc1_ref, wp1_ref, sp1_ref, bp1_ref,
               w4_ref, s4_ref, b4_ref, wc2_ref, wp2_ref, sp2_ref, bp2_ref,
               wf_ref, sf_ref, bf_ref, out_ref):
    g = g_ref[...]
    q = qtab_ref[0]
    x1 = _branch(g[:, :64], q[:, :64], _BLK, w2_ref[...], s2_ref[...],
                 b2_ref[...], wc1_ref[...], wp1_ref[...], sp1_ref[...],
                 bp1_ref[...], _K1)
    x2 = _branch(g[:, 64:], q[:, 64:], _BLK, w4_ref[...], s4_ref[...],
                 b4_ref[...], wc2_ref[...], wp2_ref[...], sp2_ref[...],
                 bp2_ref[...], _K2)
    xcat = jnp.concatenate([x1, x2], axis=1)
    out = jnp.maximum(_dgt(xcat, wf_ref[...]) * sf_ref[...] + bf_ref[...],
                      0.0)
    out_ref[...] = out.T


def _sc_gather(table, idx):
    m = idx.shape[0]
    nw = 32
    bpw = m // nw
    ch = 256
    nch = bpw // ch
    mesh = plsc.VectorSubcoreMesh(core_axis_name="c", subcore_axis_name="s")

    @functools.partial(
        pl.kernel, mesh=mesh,
        out_type=jax.ShapeDtypeStruct((m, 128), jnp.float32),
        scratch_types=[
            pltpu.VMEM((bpw,), jnp.int32),
            pltpu.VMEM((ch, 128), jnp.float32),
            pltpu.VMEM((ch, 128), jnp.float32),
            pltpu.SemaphoreType.DMA,
            pltpu.SemaphoreType.DMA,
        ],
    )
    def gather_k(table_hbm, idx_hbm, out_hbm, idx_v, rows_a, rows_b, sem_a,
                 sem_b):
        wid = lax.axis_index("s") * 2 + lax.axis_index("c")
        base = wid * bpw
        pltpu.sync_copy(idx_hbm.at[pl.ds(base, bpw)], idx_v)
        bufs = (rows_a, rows_b)
        sems = (sem_a, sem_b)
        cps = [None, None]
        for c in range(nch):
            p = c % 2
            cps[p] = pltpu.async_copy(
                table_hbm.at[idx_v.at[pl.ds(c * ch, ch)]], bufs[p], sems[p])
            if c > 0:
                q = 1 - p
                cps[q].wait()
                pltpu.sync_copy(bufs[q],
                                out_hbm.at[pl.ds(base + (c - 1) * ch, ch)])
        p = (nch - 1) % 2
        cps[p].wait()
        pltpu.sync_copy(bufs[p], out_hbm.at[pl.ds(base + (nch - 1) * ch, ch)])

    return gather_k(table, idx)


def kernel(x, params):
    p = params
    bsz, _, n = x.shape

    w1a, w1d = p['conv1_W'][:, :8], p['conv1_W'][:, 8:] - p['conv1_W'][:, :8]
    w3a, w3d = p['conv3_W'][:, :8], p['conv3_W'][:, 8:] - p['conv3_W'][:, :8]
    inv = 1.0 / jnp.sqrt(jnp.float32(1.0 + _EPS))

    def sb(g, b):
        return (g * inv)[None, :], b[None, :]

    s1v, bb1 = sb(p['bn1_g'], p['bn1_b'])
    s2v, bb2 = sb(p['bn2_g'], p['bn2_b'])
    s3v, bb3 = sb(p['bn3_g'], p['bn3_b'])
    s4v, bb4 = sb(p['bn4_g'], p['bn4_b'])
    sp1, bp1 = sb(p['att1_g'], p['att1_b'])
    sp2, bp2 = sb(p['att2_g'], p['att2_b'])
    sfv, bfv = sb(p['bn_g'], p['bn_b'])

    g1 = p['liner_g'].reshape(8, 1)
    b1 = p['liner_b'].reshape(8, 1)
    y, ytp, sq, ptab, qtab = pl.pallas_call(
        _stage1_body,
        grid=(bsz,),
        in_specs=[
            pl.BlockSpec((1, 3, n), lambda b: (b, 0, 0)),
            pl.BlockSpec((8, 3), lambda b: (0, 0)),
            pl.BlockSpec((8, 1), lambda b: (0, 0)),
            pl.BlockSpec((8, 1), lambda b: (0, 0)),
            pl.BlockSpec((64, 8), lambda b: (0, 0)),
            pl.BlockSpec((64, 8), lambda b: (0, 0)),
            pl.BlockSpec((1, 64), lambda b: (0, 0)),
            pl.BlockSpec((1, 64), lambda b: (0, 0)),
            pl.BlockSpec((64, 8), lambda b: (0, 0)),
            pl.BlockSpec((64, 8), lambda b: (0, 0)),
            pl.BlockSpec((1, 64), lambda b: (0, 0)),
            pl.BlockSpec((1, 64), lambda b: (0, 0)),
        ],
        out_specs=[
            pl.BlockSpec((1, 8, n), lambda b: (b, 0, 0)),
            pl.BlockSpec((1, n, 16), lambda b: (b, 0, 0)),
            pl.BlockSpec((1, 1, n), lambda b: (b, 0, 0)),
            pl.BlockSpec((1, n, 128), lambda b: (b, 0, 0)),
            pl.BlockSpec((1, n, 128), lambda b: (b, 0, 0)),
        ],
        out_shape=[
            jax.ShapeDtypeStruct((bsz, 8, n), jnp.float32),
            jax.ShapeDtypeStruct((bsz, n, 16), jnp.float32),
            jax.ShapeDtypeStruct((bsz, 1, n), jnp.float32),
            jax.ShapeDtypeStruct((bsz, n, 128), jnp.float32),
            jax.ShapeDtypeStruct((bsz, n, 128), jnp.float32),
        ],
    )(x, p['liner_W'], g1, b1,
      w1a, w1d, s1v, bb1, w3a, w3d, s3v, bb3)

    def wspec(shape):
        return pl.BlockSpec(shape, lambda q: tuple(0 for _ in shape))

    table = ptab.reshape(bsz * n, 128)
    outs = []
    for b in range(bsz):
        idx_b = pl.pallas_call(
            functools.partial(_knn_body, b * n),
            grid=(n // _BQ,),
            in_specs=[
                pl.BlockSpec((1, _BQ, 16), lambda q, b=b: (b, q, 0)),
                pl.BlockSpec((1, 8, n), lambda q, b=b: (b, 0, 0)),
                pl.BlockSpec((1, 1, n), lambda q, b=b: (b, 0, 0)),
            ],
            out_specs=pl.BlockSpec((_BQ, _K1), lambda q: (q, 0)),
            out_shape=jax.ShapeDtypeStruct((n, _K1), jnp.int32),
        )(ytp, y, sq)

        gathered = _sc_gather(table, idx_b.reshape(n * _K1))

        out_b = pl.pallas_call(
            _fuse_body,
            grid=(n // _BLK,),
            in_specs=[
                pl.BlockSpec((_BLK * _K1, 128), lambda q: (q, 0)),
                pl.BlockSpec((1, _BLK, 128), lambda q, b=b: (b, q, 0)),
                wspec((128, 64)), wspec((1, 128)), wspec((1, 128)),
                wspec((128, 128)), wspec((256, 128)), wspec((1, 256)),
                wspec((1, 256)),
                wspec((128, 64)), wspec((1, 128)), wspec((1, 128)),
                wspec((128, 128)), wspec((256, 128)), wspec((1, 256)),
                wspec((1, 256)),
                wspec((512, 512)), wspec((1, 512)), wspec((1, 512)),
            ],
            out_specs=pl.BlockSpec((512, _BLK), lambda q: (0, q)),
            out_shape=jax.ShapeDtypeStruct((512, n), jnp.float32),
        )(gathered, qtab,
          p['conv2_W'], s2v, bb2, p['att1_Wc'], p['att1_W2'], sp1, bp1,
          p['conv4_W'], s4v, bb4, p['att2_Wc'], p['att2_W2'], sp2, bp2,
          p['conv_W'], sfv, bfv)
        outs.append(out_b)
    return jnp.stack(outs)


# speedup vs baseline: 17.1676x; 1.0205x over previous
import functools

import jax
import jax.numpy as jnp
from jax import lax
from jax.experimental import pallas as pl
from jax.experimental.pallas import tpu as pltpu
from jax.experimental.pallas import tpu_sc as plsc

_EPS = 1e-5
_K1 = 16
_K2 = 12
_BQ = 256
_BLK = 256


def _dgt(a, b):
    return lax.dot_general(a, b, (((1,), (1,)), ((), ())),
                           preferred_element_type=jnp.float32)


def _stage1_body(x_ref, w_ref, g_ref, b_ref,
                 w1a_ref, w1d_ref, s1_ref, bb1_ref,
                 w3a_ref, w3d_ref, s3_ref, bb3_ref,
                 y_ref, ytp_ref, sq_ref, ptab_ref, qtab_ref):
    x = x_ref[0]
    w = w_ref[...]
    y = lax.dot_general(w, x, (((1,), (0,)), ((), ())),
                        preferred_element_type=jnp.float32)
    scale = g_ref[...] / jnp.sqrt(1.0 + _EPS)
    y = y * scale + b_ref[...]
    y = jnp.maximum(y, 0.0)
    y_ref[0] = y
    sq_ref[0] = jnp.sum(y * y, axis=0, keepdims=True)
    yt = y.T
    ytp_ref[0] = jnp.concatenate([yt, jnp.zeros_like(yt)], axis=1)
    s1 = s1_ref[...]
    s3 = s3_ref[...]
    p1 = _dgt(yt, w1a_ref[...]) * s1
    p3 = _dgt(yt, w3a_ref[...]) * s3
    ptab_ref[0] = jnp.concatenate([p1, p3], axis=1)
    q1 = _dgt(yt, w1d_ref[...]) * s1 + bb1_ref[...]
    q3 = _dgt(yt, w3d_ref[...]) * s3 + bb3_ref[...]
    qtab_ref[0] = jnp.concatenate([q1, q3], axis=1)


def _knn_body(base, ytp_ref, y_ref, sq_ref, idx_ref):
    n = y_ref.shape[2]
    yq = ytp_ref[0][:, :8]
    yk = y_ref[0]
    sqk = sq_ref[0]
    d = 2.0 * lax.dot_general(yq, yk, (((1,), (0,)), ((), ())),
                              preferred_element_type=jnp.float32)
    d = d - sqk
    niota = (float(n)
             - lax.broadcasted_iota(jnp.int32, d.shape, 1).astype(jnp.float32))
    cols = []
    for _ in range(_K1):
        m = jnp.max(d, axis=1, keepdims=True)
        c = jnp.where(d == m, niota, 0.0)
        mx = jnp.max(c, axis=1, keepdims=True)
        cols.append(mx)
        d = jnp.where(niota == mx, -jnp.inf, d)
    idxf = float(n) - jnp.concatenate(cols, axis=1)
    idx_ref[...] = idxf.astype(jnp.int32) + base


def _branch(n1, q1, blk, w2, s2, b2, wc, wp, sp, bp, kmax):
    h1 = jnp.maximum(n1.reshape(blk, _K1, 64) + q1[:, None, :], 0.0)
    h1 = h1.reshape(blk * _K1, 64)
    h2 = jnp.maximum(_dgt(h1, w2) * s2 + b2, 0.0)
    s = jax.nn.sigmoid(_dgt(h2, wc))
    ps = (h2 * s).reshape(blk, _K1, 128)
    if kmax < _K1:
        slot = lax.broadcasted_iota(jnp.int32, ps.shape, 1)
        ps = jnp.where(slot < kmax, ps, 0.0)
    pooled = jnp.sum(ps, axis=1)
    return jnp.maximum(_dgt(pooled, wp) * sp + bp, 0.0)


def _fuse_body(g_ref, qtab_ref,
               w2_ref, s2_ref, b2_ref, wc1_ref, wp1_ref, sp1_ref, bp1_ref,
               w4_ref, s4_ref, b4_ref, wc2_ref, wp2_ref, sp2_ref, bp2_ref,
               wf_ref, sf_ref, bf_ref, *rest):
    out_ref = rest[-1]
    g = g_ref[...]
    q = qtab_ref[0]
    x1 = _branch(g[:, :64], q[:, :64], _BLK, w2_ref[...], s2_ref[...],
                 b2_ref[...], wc1_ref[...], wp1_ref[...], sp1_ref[...],
                 bp1_ref[...], _K1)
    x2 = _branch(g[:, 64:], q[:, 64:], _BLK, w4_ref[...], s4_ref[...],
                 b4_ref[...], wc2_ref[...], wp2_ref[...], sp2_ref[...],
                 bp2_ref[...], _K2)
    xcat = jnp.concatenate([x1, x2], axis=1)
    out = jnp.maximum(_dgt(xcat, wf_ref[...]) * sf_ref[...] + bf_ref[...],
                      0.0)
    out_ref[0] = out.T


def _sc_gather(table, idx):
    m = idx.shape[0]
    nw = 32
    bpw = m // nw
    ch = 256
    nch = bpw // ch
    mesh = plsc.VectorSubcoreMesh(core_axis_name="c", subcore_axis_name="s")

    @functools.partial(
        pl.kernel, mesh=mesh,
        out_type=jax.ShapeDtypeStruct((m, 128), jnp.float32),
        scratch_types=[
            pltpu.VMEM((bpw,), jnp.int32),
            pltpu.VMEM((ch, 128), jnp.float32),
            pltpu.VMEM((ch, 128), jnp.float32),
            pltpu.SemaphoreType.DMA,
            pltpu.SemaphoreType.DMA,
        ],
    )
    def gather_k(table_hbm, idx_hbm, out_hbm, idx_v, rows_a, rows_b, sem_a,
                 sem_b):
        wid = lax.axis_index("s") * 2 + lax.axis_index("c")
        base = wid * bpw
        pltpu.sync_copy(idx_hbm.at[pl.ds(base, bpw)], idx_v)
        bufs = (rows_a, rows_b)
        sems = (sem_a, sem_b)
        cps = [None, None]
        for c in range(nch):
            p = c % 2
            cps[p] = pltpu.async_copy(
                table_hbm.at[idx_v.at[pl.ds(c * ch, ch)]], bufs[p], sems[p])
            if c > 0:
                q = 1 - p
                cps[q].wait()
                pltpu.sync_copy(bufs[q],
                                out_hbm.at[pl.ds(base + (c - 1) * ch, ch)])
        p = (nch - 1) % 2
        cps[p].wait()
        pltpu.sync_copy(bufs[p], out_hbm.at[pl.ds(base + (nch - 1) * ch, ch)])

    return gather_k(table, idx)


def kernel(x, params):
    p = params
    bsz, _, n = x.shape

    w1a, w1d = p['conv1_W'][:, :8], p['conv1_W'][:, 8:] - p['conv1_W'][:, :8]
    w3a, w3d = p['conv3_W'][:, :8], p['conv3_W'][:, 8:] - p['conv3_W'][:, :8]
    inv = 1.0 / jnp.sqrt(jnp.float32(1.0 + _EPS))

    def sb(g, b):
        return (g * inv)[None, :], b[None, :]

    s1v, bb1 = sb(p['bn1_g'], p['bn1_b'])
    s2v, bb2 = sb(p['bn2_g'], p['bn2_b'])
    s3v, bb3 = sb(p['bn3_g'], p['bn3_b'])
    s4v, bb4 = sb(p['bn4_g'], p['bn4_b'])
    sp1, bp1 = sb(p['att1_g'], p['att1_b'])
    sp2, bp2 = sb(p['att2_g'], p['att2_b'])
    sfv, bfv = sb(p['bn_g'], p['bn_b'])

    g1 = p['liner_g'].reshape(8, 1)
    b1 = p['liner_b'].reshape(8, 1)
    y, ytp, sq, ptab, qtab = pl.pallas_call(
        _stage1_body,
        grid=(bsz,),
        in_specs=[
            pl.BlockSpec((1, 3, n), lambda b: (b, 0, 0)),
            pl.BlockSpec((8, 3), lambda b: (0, 0)),
            pl.BlockSpec((8, 1), lambda b: (0, 0)),
            pl.BlockSpec((8, 1), lambda b: (0, 0)),
            pl.BlockSpec((64, 8), lambda b: (0, 0)),
            pl.BlockSpec((64, 8), lambda b: (0, 0)),
            pl.BlockSpec((1, 64), lambda b: (0, 0)),
            pl.BlockSpec((1, 64), lambda b: (0, 0)),
            pl.BlockSpec((64, 8), lambda b: (0, 0)),
            pl.BlockSpec((64, 8), lambda b: (0, 0)),
            pl.BlockSpec((1, 64), lambda b: (0, 0)),
            pl.BlockSpec((1, 64), lambda b: (0, 0)),
        ],
        out_specs=[
            pl.BlockSpec((1, 8, n), lambda b: (b, 0, 0)),
            pl.BlockSpec((1, n, 16), lambda b: (b, 0, 0)),
            pl.BlockSpec((1, 1, n), lambda b: (b, 0, 0)),
            pl.BlockSpec((1, n, 128), lambda b: (b, 0, 0)),
            pl.BlockSpec((1, n, 128), lambda b: (b, 0, 0)),
        ],
        out_shape=[
            jax.ShapeDtypeStruct((bsz, 8, n), jnp.float32),
            jax.ShapeDtypeStruct((bsz, n, 16), jnp.float32),
            jax.ShapeDtypeStruct((bsz, 1, n), jnp.float32),
            jax.ShapeDtypeStruct((bsz, n, 128), jnp.float32),
            jax.ShapeDtypeStruct((bsz, n, 128), jnp.float32),
        ],
    )(x, p['liner_W'], g1, b1,
      w1a, w1d, s1v, bb1, w3a, w3d, s3v, bb3)

    def wspec(shape):
        return pl.BlockSpec(shape, lambda q: tuple(0 for _ in shape))

    table = ptab.reshape(bsz * n, 128)
    acc = None
    for b in range(bsz):
        idx_b = pl.pallas_call(
            functools.partial(_knn_body, b * n),
            grid=(n // _BQ,),
            in_specs=[
                pl.BlockSpec((1, _BQ, 16), lambda q, b=b: (b, q, 0)),
                pl.BlockSpec((1, 8, n), lambda q, b=b: (b, 0, 0)),
                pl.BlockSpec((1, 1, n), lambda q, b=b: (b, 0, 0)),
            ],
            out_specs=pl.BlockSpec((_BQ, _K1), lambda q: (q, 0)),
            out_shape=jax.ShapeDtypeStruct((n, _K1), jnp.int32),
        )(ytp, y, sq)

        gathered = _sc_gather(table, idx_b.reshape(n * _K1))

        in_specs = [
            pl.BlockSpec((_BLK * _K1, 128), lambda q: (q, 0)),
            pl.BlockSpec((1, _BLK, 128), lambda q, b=b: (b, q, 0)),
            wspec((128, 64)), wspec((1, 128)), wspec((1, 128)),
            wspec((128, 128)), wspec((256, 128)), wspec((1, 256)),
            wspec((1, 256)),
            wspec((128, 64)), wspec((1, 128)), wspec((1, 128)),
            wspec((128, 128)), wspec((256, 128)), wspec((1, 256)),
            wspec((1, 256)),
            wspec((512, 512)), wspec((1, 512)), wspec((1, 512)),
        ]
        args = [gathered, qtab,
                p['conv2_W'], s2v, bb2, p['att1_Wc'], p['att1_W2'], sp1, bp1,
                p['conv4_W'], s4v, bb4, p['att2_Wc'], p['att2_W2'], sp2, bp2,
                p['conv_W'], sfv, bfv]
        aliases = {}
        if b > 0:
            in_specs.append(pl.BlockSpec((1, 8, _BLK), lambda q: (0, 0, q)))
            args.append(acc)
            aliases = {len(in_specs) - 1: 0}
        acc = pl.pallas_call(
            _fuse_body,
            grid=(n // _BLK,),
            in_specs=in_specs,
            out_specs=pl.BlockSpec((1, 512, _BLK), lambda q, b=b: (b, 0, q)),
            out_shape=jax.ShapeDtypeStruct((bsz, 512, n), jnp.float32),
            input_output_aliases=aliases,
        )(*args)
    return acc


# speedup vs baseline: 17.2195x; 1.0030x over previous
import functools

import jax
import jax.numpy as jnp
from jax import lax
from jax.experimental import pallas as pl
from jax.experimental.pallas import tpu as pltpu
from jax.experimental.pallas import tpu_sc as plsc

_EPS = 1e-5
_K1 = 16
_K2 = 12
_BQ = 256
_BLK = 256


def _dgt(a, b):
    return lax.dot_general(a, b, (((1,), (1,)), ((), ())),
                           preferred_element_type=jnp.float32)


def _stage1_body(x_ref, w_ref, g_ref, b_ref,
                 w1_ref, g1b_ref, b1b_ref,
                 w3_ref, g3b_ref, b3b_ref,
                 y_ref, ytp_ref, sq_ref, ptab_ref, qtab_ref):
    x = x_ref[0]
    w = w_ref[...]
    y = lax.dot_general(w, x, (((1,), (0,)), ((), ())),
                        preferred_element_type=jnp.float32)
    scale = g_ref[...] / jnp.sqrt(1.0 + _EPS)
    y = y * scale + b_ref[...]
    y = jnp.maximum(y, 0.0)
    y_ref[0] = y
    sq_ref[0] = jnp.sum(y * y, axis=0, keepdims=True)
    yt = y.T
    ytp_ref[0] = jnp.concatenate([yt, jnp.zeros_like(yt)], axis=1)
    w1 = w1_ref[...]
    w3 = w3_ref[...]
    w1a, w1d = w1[:, :8], w1[:, 8:] - w1[:, :8]
    w3a, w3d = w3[:, :8], w3[:, 8:] - w3[:, :8]
    s1 = g1b_ref[...] / jnp.sqrt(1.0 + _EPS)
    s3 = g3b_ref[...] / jnp.sqrt(1.0 + _EPS)
    p1 = _dgt(yt, w1a) * s1
    p3 = _dgt(yt, w3a) * s3
    ptab_ref[0] = jnp.concatenate([p1, p3], axis=1)
    q1 = _dgt(yt, w1d) * s1 + b1b_ref[...]
    q3 = _dgt(yt, w3d) * s3 + b3b_ref[...]
    qtab_ref[0] = jnp.concatenate([q1, q3], axis=1)


def _knn_body(base, ytp_ref, y_ref, sq_ref, idx_ref):
    n = y_ref.shape[2]
    yq = ytp_ref[0][:, :8]
    yk = y_ref[0]
    sqk = sq_ref[0]
    d = 2.0 * lax.dot_general(yq, yk, (((1,), (0,)), ((), ())),
                              preferred_element_type=jnp.float32)
    d = d - sqk
    niota = (float(n)
             - lax.broadcasted_iota(jnp.int32, d.shape, 1).astype(jnp.float32))
    cols = []
    for _ in range(_K1):
        m = jnp.max(d, axis=1, keepdims=True)
        c = jnp.where(d == m, niota, 0.0)
        mx = jnp.max(c, axis=1, keepdims=True)
        cols.append(mx)
        d = jnp.where(niota == mx, -jnp.inf, d)
    idxf = float(n) - jnp.concatenate(cols, axis=1)
    idx_ref[...] = idxf.astype(jnp.int32) + base


def _branch(n1, q1, blk, w2, g2, b2, wc, wp, gp, bp, kmax):
    s2 = g2 / jnp.sqrt(1.0 + _EPS)
    sp = gp / jnp.sqrt(1.0 + _EPS)
    h1 = jnp.maximum(n1.reshape(blk, _K1, 64) + q1[:, None, :], 0.0)
    h1 = h1.reshape(blk * _K1, 64)
    h2 = jnp.maximum(_dgt(h1, w2) * s2 + b2, 0.0)
    s = jax.nn.sigmoid(_dgt(h2, wc))
    ps = (h2 * s).reshape(blk, _K1, 128)
    if kmax < _K1:
        slot = lax.broadcasted_iota(jnp.int32, ps.shape, 1)
        ps = jnp.where(slot < kmax, ps, 0.0)
    pooled = jnp.sum(ps, axis=1)
    return jnp.maximum(_dgt(pooled, wp) * sp + bp, 0.0)


def _fuse_body(g_ref, qtab_ref,
               w2_ref, s2_ref, b2_ref, wc1_ref, wp1_ref, sp1_ref, bp1_ref,
               w4_ref, s4_ref, b4_ref, wc2_ref, wp2_ref, sp2_ref, bp2_ref,
               wf_ref, sf_ref, bf_ref, *rest):
    out_ref = rest[-1]
    g = g_ref[...]
    q = qtab_ref[0]
    x1 = _branch(g[:, :64], q[:, :64], _BLK, w2_ref[...], s2_ref[...],
                 b2_ref[...], wc1_ref[...], wp1_ref[...], sp1_ref[...],
                 bp1_ref[...], _K1)
    x2 = _branch(g[:, 64:], q[:, 64:], _BLK, w4_ref[...], s4_ref[...],
                 b4_ref[...], wc2_ref[...], wp2_ref[...], sp2_ref[...],
                 bp2_ref[...], _K2)
    xcat = jnp.concatenate([x1, x2], axis=1)
    sf = sf_ref[...] / jnp.sqrt(1.0 + _EPS)
    out = jnp.maximum(_dgt(xcat, wf_ref[...]) * sf + bf_ref[...], 0.0)
    out_ref[0] = out.T


def _sc_gather(table, idx):
    m = idx.shape[0]
    nw = 32
    bpw = m // nw
    ch = 256
    nch = bpw // ch
    mesh = plsc.VectorSubcoreMesh(core_axis_name="c", subcore_axis_name="s")

    @functools.partial(
        pl.kernel, mesh=mesh,
        out_type=jax.ShapeDtypeStruct((m, 128), jnp.float32),
        scratch_types=[
            pltpu.VMEM((bpw,), jnp.int32),
            pltpu.VMEM((ch, 128), jnp.float32),
            pltpu.VMEM((ch, 128), jnp.float32),
            pltpu.SemaphoreType.DMA,
            pltpu.SemaphoreType.DMA,
        ],
    )
    def gather_k(table_hbm, idx_hbm, out_hbm, idx_v, rows_a, rows_b, sem_a,
                 sem_b):
        wid = lax.axis_index("s") * 2 + lax.axis_index("c")
        base = wid * bpw
        pltpu.sync_copy(idx_hbm.at[pl.ds(base, bpw)], idx_v)
        bufs = (rows_a, rows_b)
        sems = (sem_a, sem_b)
        cps = [None, None]
        for c in range(nch):
            p = c % 2
            cps[p] = pltpu.async_copy(
                table_hbm.at[idx_v.at[pl.ds(c * ch, ch)]], bufs[p], sems[p])
            if c > 0:
                q = 1 - p
                cps[q].wait()
                pltpu.sync_copy(bufs[q],
                                out_hbm.at[pl.ds(base + (c - 1) * ch, ch)])
        p = (nch - 1) % 2
        cps[p].wait()
        pltpu.sync_copy(bufs[p], out_hbm.at[pl.ds(base + (nch - 1) * ch, ch)])

    return gather_k(table, idx)


def kernel(x, params):
    p = params
    bsz, _, n = x.shape

    def rs(v):
        return v.reshape(1, -1)

    g1 = p['liner_g'].reshape(8, 1)
    b1 = p['liner_b'].reshape(8, 1)
    y, ytp, sq, ptab, qtab = pl.pallas_call(
        _stage1_body,
        grid=(bsz,),
        in_specs=[
            pl.BlockSpec((1, 3, n), lambda b: (b, 0, 0)),
            pl.BlockSpec((8, 3), lambda b: (0, 0)),
            pl.BlockSpec((8, 1), lambda b: (0, 0)),
            pl.BlockSpec((8, 1), lambda b: (0, 0)),
            pl.BlockSpec((64, 16), lambda b: (0, 0)),
            pl.BlockSpec((1, 64), lambda b: (0, 0)),
            pl.BlockSpec((1, 64), lambda b: (0, 0)),
            pl.BlockSpec((64, 16), lambda b: (0, 0)),
            pl.BlockSpec((1, 64), lambda b: (0, 0)),
            pl.BlockSpec((1, 64), lambda b: (0, 0)),
        ],
        out_specs=[
            pl.BlockSpec((1, 8, n), lambda b: (b, 0, 0)),
            pl.BlockSpec((1, n, 16), lambda b: (b, 0, 0)),
            pl.BlockSpec((1, 1, n), lambda b: (b, 0, 0)),
            pl.BlockSpec((1, n, 128), lambda b: (b, 0, 0)),
            pl.BlockSpec((1, n, 128), lambda b: (b, 0, 0)),
        ],
        out_shape=[
            jax.ShapeDtypeStruct((bsz, 8, n), jnp.float32),
            jax.ShapeDtypeStruct((bsz, n, 16), jnp.float32),
            jax.ShapeDtypeStruct((bsz, 1, n), jnp.float32),
            jax.ShapeDtypeStruct((bsz, n, 128), jnp.float32),
            jax.ShapeDtypeStruct((bsz, n, 128), jnp.float32),
        ],
    )(x, p['liner_W'], g1, b1,
      p['conv1_W'], rs(p['bn1_g']), rs(p['bn1_b']),
      p['conv3_W'], rs(p['bn3_g']), rs(p['bn3_b']))

    def wspec(shape):
        return pl.BlockSpec(shape, lambda q: tuple(0 for _ in shape))

    table = ptab.reshape(bsz * n, 128)
    acc = None
    for b in range(bsz):
        idx_b = pl.pallas_call(
            functools.partial(_knn_body, b * n),
            grid=(n // _BQ,),
            in_specs=[
                pl.BlockSpec((1, _BQ, 16), lambda q, b=b: (b, q, 0)),
                pl.BlockSpec((1, 8, n), lambda q, b=b: (b, 0, 0)),
                pl.BlockSpec((1, 1, n), lambda q, b=b: (b, 0, 0)),
            ],
            out_specs=pl.BlockSpec((_BQ, _K1), lambda q: (q, 0)),
            out_shape=jax.ShapeDtypeStruct((n, _K1), jnp.int32),
        )(ytp, y, sq)

        gathered = _sc_gather(table, idx_b.reshape(n * _K1))

        in_specs = [
            pl.BlockSpec((_BLK * _K1, 128), lambda q: (q, 0)),
            pl.BlockSpec((1, _BLK, 128), lambda q, b=b: (b, q, 0)),
            wspec((128, 64)), wspec((1, 128)), wspec((1, 128)),
            wspec((128, 128)), wspec((256, 128)), wspec((1, 256)),
            wspec((1, 256)),
            wspec((128, 64)), wspec((1, 128)), wspec((1, 128)),
            wspec((128, 128)), wspec((256, 128)), wspec((1, 256)),
            wspec((1, 256)),
            wspec((512, 512)), wspec((1, 512)), wspec((1, 512)),
        ]
        args = [gathered, qtab,
                p['conv2_W'], rs(p['bn2_g']), rs(p['bn2_b']),
                p['att1_Wc'], p['att1_W2'], rs(p['att1_g']), rs(p['att1_b']),
                p['conv4_W'], rs(p['bn4_g']), rs(p['bn4_b']),
                p['att2_Wc'], p['att2_W2'], rs(p['att2_g']), rs(p['att2_b']),
                p['conv_W'], rs(p['bn_g']), rs(p['bn_b'])]
        aliases = {}
        if b > 0:
            in_specs.append(pl.BlockSpec((1, 8, _BLK), lambda q: (0, 0, q)))
            args.append(acc)
            aliases = {len(in_specs) - 1: 0}
        acc = pl.pallas_call(
            _fuse_body,
            grid=(n // _BLK,),
            in_specs=in_specs,
            out_specs=pl.BlockSpec((1, 512, _BLK), lambda q, b=b: (b, 0, q)),
            out_shape=jax.ShapeDtypeStruct((bsz, 512, n), jnp.float32),
            input_output_aliases=aliases,
        )(*args)
    return acc


# speedup vs baseline: 17.4006x; 1.0105x over previous
import functools

import jax
import jax.numpy as jnp
from jax import lax
from jax.experimental import pallas as pl
from jax.experimental.pallas import tpu as pltpu
from jax.experimental.pallas import tpu_sc as plsc

_EPS = 1e-5
_K1 = 16
_K2 = 12
_BQ = 256
_BLK = 512


def _dgt(a, b):
    return lax.dot_general(a, b, (((1,), (1,)), ((), ())),
                           preferred_element_type=jnp.float32)


def _stage1_body(x_ref, w_ref, g_ref, b_ref,
                 w1_ref, g1b_ref, b1b_ref,
                 w3_ref, g3b_ref, b3b_ref,
                 y_ref, ytp_ref, sq_ref, ptab_ref, qtab_ref):
    x = x_ref[0]
    w = w_ref[...]
    y = lax.dot_general(w, x, (((1,), (0,)), ((), ())),
                        preferred_element_type=jnp.float32)
    scale = g_ref[...] / jnp.sqrt(1.0 + _EPS)
    y = y * scale + b_ref[...]
    y = jnp.maximum(y, 0.0)
    y_ref[0] = y
    sq_ref[0] = jnp.sum(y * y, axis=0, keepdims=True)
    yt = y.T
    ytp_ref[0] = jnp.concatenate([yt, jnp.zeros_like(yt)], axis=1)
    w1 = w1_ref[...]
    w3 = w3_ref[...]
    w1a, w1d = w1[:, :8], w1[:, 8:] - w1[:, :8]
    w3a, w3d = w3[:, :8], w3[:, 8:] - w3[:, :8]
    s1 = g1b_ref[...] / jnp.sqrt(1.0 + _EPS)
    s3 = g3b_ref[...] / jnp.sqrt(1.0 + _EPS)
    p1 = _dgt(yt, w1a) * s1
    p3 = _dgt(yt, w3a) * s3
    ptab_ref[0] = jnp.concatenate([p1, p3], axis=1)
    q1 = _dgt(yt, w1d) * s1 + b1b_ref[...]
    q3 = _dgt(yt, w3d) * s3 + b3b_ref[...]
    qtab_ref[0] = jnp.concatenate([q1, q3], axis=1)


def _knn_body(base, ytp_ref, y_ref, sq_ref, idx_ref):
    n = y_ref.shape[2]
    yq = ytp_ref[0][:, :8]
    yk = y_ref[0]
    sqk = sq_ref[0]
    d = 2.0 * lax.dot_general(yq, yk, (((1,), (0,)), ((), ())),
                              preferred_element_type=jnp.float32)
    d = d - sqk
    niota = (float(n)
             - lax.broadcasted_iota(jnp.int32, d.shape, 1).astype(jnp.float32))
    cols = []
    for _ in range(_K1):
        m = jnp.max(d, axis=1, keepdims=True)
        c = jnp.where(d == m, niota, 0.0)
        mx = jnp.max(c, axis=1, keepdims=True)
        cols.append(mx)
        d = jnp.where(niota == mx, -jnp.inf, d)
    idxf = float(n) - jnp.concatenate(cols, axis=1)
    idx_ref[...] = idxf.astype(jnp.int32) + base


def _branch(n1, q1, blk, w2, g2, b2, wc, wp, gp, bp, kmax):
    s2 = g2 / jnp.sqrt(1.0 + _EPS)
    sp = gp / jnp.sqrt(1.0 + _EPS)
    h1 = jnp.maximum(n1.reshape(blk, _K1, 64) + q1[:, None, :], 0.0)
    h1 = h1.reshape(blk * _K1, 64)
    h2 = jnp.maximum(_dgt(h1, w2) * s2 + b2, 0.0)
    s = jax.nn.sigmoid(_dgt(h2, wc))
    ps = (h2 * s).reshape(blk, _K1, 128)
    if kmax < _K1:
        slot = lax.broadcasted_iota(jnp.int32, ps.shape, 1)
        ps = jnp.where(slot < kmax, ps, 0.0)
    pooled = jnp.sum(ps, axis=1)
    return jnp.maximum(_dgt(pooled, wp) * sp + bp, 0.0)


def _fuse_body(g_ref, qtab_ref,
               w2_ref, s2_ref, b2_ref, wc1_ref, wp1_ref, sp1_ref, bp1_ref,
               w4_ref, s4_ref, b4_ref, wc2_ref, wp2_ref, sp2_ref, bp2_ref,
               wf_ref, sf_ref, bf_ref, *rest):
    out_ref = rest[-1]
    g = g_ref[...]
    q = qtab_ref[0]
    x1 = _branch(g[:, :64], q[:, :64], _BLK, w2_ref[...], s2_ref[...],
                 b2_ref[...], wc1_ref[...], wp1_ref[...], sp1_ref[...],
                 bp1_ref[...], _K1)
    x2 = _branch(g[:, 64:], q[:, 64:], _BLK, w4_ref[...], s4_ref[...],
                 b4_ref[...], wc2_ref[...], wp2_ref[...], sp2_ref[...],
                 bp2_ref[...], _K2)
    xcat = jnp.concatenate([x1, x2], axis=1)
    sf = sf_ref[...] / jnp.sqrt(1.0 + _EPS)
    out = jnp.maximum(_dgt(xcat, wf_ref[...]) * sf + bf_ref[...], 0.0)
    out_ref[0] = out.T


def _sc_gather(table, idx):
    m = idx.shape[0]
    nw = 32
    bpw = m // nw
    ch = 256
    nch = bpw // ch
    mesh = plsc.VectorSubcoreMesh(core_axis_name="c", subcore_axis_name="s")

    @functools.partial(
        pl.kernel, mesh=mesh,
        out_type=jax.ShapeDtypeStruct((m, 128), jnp.float32),
        scratch_types=[
            pltpu.VMEM((bpw,), jnp.int32),
            pltpu.VMEM((ch, 128), jnp.float32),
            pltpu.VMEM((ch, 128), jnp.float32),
            pltpu.SemaphoreType.DMA,
            pltpu.SemaphoreType.DMA,
        ],
    )
    def gather_k(table_hbm, idx_hbm, out_hbm, idx_v, rows_a, rows_b, sem_a,
                 sem_b):
        wid = lax.axis_index("s") * 2 + lax.axis_index("c")
        base = wid * bpw
        pltpu.sync_copy(idx_hbm.at[pl.ds(base, bpw)], idx_v)
        bufs = (rows_a, rows_b)
        sems = (sem_a, sem_b)
        cps = [None, None]
        for c in range(nch):
            p = c % 2
            cps[p] = pltpu.async_copy(
                table_hbm.at[idx_v.at[pl.ds(c * ch, ch)]], bufs[p], sems[p])
            if c > 0:
                q = 1 - p
                cps[q].wait()
                pltpu.sync_copy(bufs[q],
                                out_hbm.at[pl.ds(base + (c - 1) * ch, ch)])
        p = (nch - 1) % 2
        cps[p].wait()
        pltpu.sync_copy(bufs[p], out_hbm.at[pl.ds(base + (nch - 1) * ch, ch)])

    return gather_k(table, idx)


def kernel(x, params):
    p = params
    bsz, _, n = x.shape

    def rs(v):
        return v.reshape(1, -1)

    g1 = p['liner_g'].reshape(8, 1)
    b1 = p['liner_b'].reshape(8, 1)
    y, ytp, sq, ptab, qtab = pl.pallas_call(
        _stage1_body,
        grid=(bsz,),
        in_specs=[
            pl.BlockSpec((1, 3, n), lambda b: (b, 0, 0)),
            pl.BlockSpec((8, 3), lambda b: (0, 0)),
            pl.BlockSpec((8, 1), lambda b: (0, 0)),
            pl.BlockSpec((8, 1), lambda b: (0, 0)),
            pl.BlockSpec((64, 16), lambda b: (0, 0)),
            pl.BlockSpec((1, 64), lambda b: (0, 0)),
            pl.BlockSpec((1, 64), lambda b: (0, 0)),
            pl.BlockSpec((64, 16), lambda b: (0, 0)),
            pl.BlockSpec((1, 64), lambda b: (0, 0)),
            pl.BlockSpec((1, 64), lambda b: (0, 0)),
        ],
        out_specs=[
            pl.BlockSpec((1, 8, n), lambda b: (b, 0, 0)),
            pl.BlockSpec((1, n, 16), lambda b: (b, 0, 0)),
            pl.BlockSpec((1, 1, n), lambda b: (b, 0, 0)),
            pl.BlockSpec((1, n, 128), lambda b: (b, 0, 0)),
            pl.BlockSpec((1, n, 128), lambda b: (b, 0, 0)),
        ],
        out_shape=[
            jax.ShapeDtypeStruct((bsz, 8, n), jnp.float32),
            jax.ShapeDtypeStruct((bsz, n, 16), jnp.float32),
            jax.ShapeDtypeStruct((bsz, 1, n), jnp.float32),
            jax.ShapeDtypeStruct((bsz, n, 128), jnp.float32),
            jax.ShapeDtypeStruct((bsz, n, 128), jnp.float32),
        ],
    )(x, p['liner_W'], g1, b1,
      p['conv1_W'], rs(p['bn1_g']), rs(p['bn1_b']),
      p['conv3_W'], rs(p['bn3_g']), rs(p['bn3_b']))

    def wspec(shape):
        return pl.BlockSpec(shape, lambda q: tuple(0 for _ in shape))

    table = ptab.reshape(bsz * n, 128)
    acc = None
    for b in range(bsz):
        idx_b = pl.pallas_call(
            functools.partial(_knn_body, b * n),
            grid=(n // _BQ,),
            in_specs=[
                pl.BlockSpec((1, _BQ, 16), lambda q, b=b: (b, q, 0)),
                pl.BlockSpec((1, 8, n), lambda q, b=b: (b, 0, 0)),
                pl.BlockSpec((1, 1, n), lambda q, b=b: (b, 0, 0)),
            ],
            out_specs=pl.BlockSpec((_BQ, _K1), lambda q: (q, 0)),
            out_shape=jax.ShapeDtypeStruct((n, _K1), jnp.int32),
        )(ytp, y, sq)

        gathered = _sc_gather(table, idx_b.reshape(n * _K1))

        in_specs = [
            pl.BlockSpec((_BLK * _K1, 128), lambda q: (q, 0)),
            pl.BlockSpec((1, _BLK, 128), lambda q, b=b: (b, q, 0)),
            wspec((128, 64)), wspec((1, 128)), wspec((1, 128)),
            wspec((128, 128)), wspec((256, 128)), wspec((1, 256)),
            wspec((1, 256)),
            wspec((128, 64)), wspec((1, 128)), wspec((1, 128)),
            wspec((128, 128)), wspec((256, 128)), wspec((1, 256)),
            wspec((1, 256)),
            wspec((512, 512)), wspec((1, 512)), wspec((1, 512)),
        ]
        args = [gathered, qtab,
                p['conv2_W'], rs(p['bn2_g']), rs(p['bn2_b']),
                p['att1_Wc'], p['att1_W2'], rs(p['att1_g']), rs(p['att1_b']),
                p['conv4_W'], rs(p['bn4_g']), rs(p['bn4_b']),
                p['att2_Wc'], p['att2_W2'], rs(p['att2_g']), rs(p['att2_b']),
                p['conv_W'], rs(p['bn_g']), rs(p['bn_b'])]
        aliases = {}
        if b > 0:
            in_specs.append(pl.BlockSpec((1, 8, _BLK), lambda q: (0, 0, q)))
            args.append(acc)
            aliases = {len(in_specs) - 1: 0}
        acc = pl.pallas_call(
            _fuse_body,
            grid=(n // _BLK,),
            in_specs=in_specs,
            out_specs=pl.BlockSpec((1, 512, _BLK), lambda q, b=b: (b, 0, q)),
            out_shape=jax.ShapeDtypeStruct((bsz, 512, n), jnp.float32),
            input_output_aliases=aliases,
        )(*args)
    return acc
